# baseline (speedup 1.0000x reference)
import functools

import jax
import jax.numpy as jnp
from jax import lax
from jax.experimental import pallas as pl
from jax.experimental.pallas import tpu as pltpu
from jax.experimental.pallas import tpu_sc as plsc

TOPK = 5
RUN = 8
NEG = float("-inf")
IMAX = 2**31 - 1


BIG = 1.0e8


def _tc_body(q_ref, k_ref, vals_ref, idx_ref, wlo_ref, whi_ref, i2_ref,
             qn_sc, rv_sc, ri_sc):
    step = pl.program_id(0)
    nsteps = pl.num_programs(0)
    kb = k_ref.shape[0]

    @pl.when(step == 0)
    def _():
        rv_sc[...] = jnp.full(rv_sc.shape, NEG, jnp.float32)
        ri_sc[...] = jnp.full(ri_sc.shape, BIG, jnp.float32)
        q = q_ref[...]
        qn_sc[...] = q / (jnp.sqrt(jnp.sum(q * q, axis=-1, keepdims=True))
                          + 1e-8)

    k = k_ref[...]
    kn = k / (jnp.sqrt(jnp.sum(k * k, axis=-1, keepdims=True)) + 1e-8)
    s = lax.dot_general(qn_sc[...], kn, (((1,), (1,)), ((), ())),
                        preferred_element_type=jnp.float32)
    fl = lax.broadcasted_iota(jnp.int32, s.shape, 1).astype(jnp.float32)
    base = (step * kb).astype(jnp.float32)
    rv0 = rv_sc[...]
    ri0 = ri_sc[...]

    def rounds(exact):
        x, rv = s, rv0
        vs, is_ = [], []
        for _ in range(TOPK):
            m = jnp.maximum(jnp.max(x, axis=-1, keepdims=True),
                            jnp.max(rv, axis=-1, keepdims=True))
            eb = x == m
            er = rv == m
            ib = jnp.min(jnp.where(eb, fl, BIG), axis=-1, keepdims=True) + base
            ir = jnp.min(jnp.where(er, ri0, BIG), axis=-1, keepdims=True)
            i = jnp.minimum(ib, ir)
            if exact:
                x = jnp.where(fl == i - base, NEG, x)
                rv = jnp.where(ri0 == i, NEG, rv)
            else:
                x = jnp.where(eb, NEG, x)
                rv = jnp.where(er, NEG, rv)
            vs.append(m)
            is_.append(i)
        out = (jnp.concatenate(vs, axis=-1), jnp.concatenate(is_, axis=-1))
        return (out, x, rv) if not exact else out

    (fv, fi), xf, rvf = rounds(exact=False)
    removed = (jnp.sum(jnp.where(xf == NEG, 1.0, 0.0))
               + jnp.sum(jnp.where(rvf == NEG, 1.0, 0.0)))
    qq = s.shape[0]
    expected = jnp.where(step == 0, float(qq * (TOPK + RUN)),
                         float(qq * (TOPK + RUN - TOPK)))
    mv, mi = lax.cond(removed == expected,
                      lambda: (fv, fi), lambda: rounds(exact=True))

    rv_sc[:, :TOPK] = mv
    ri_sc[:, :TOPK] = mi

    @pl.when(step == nsteps - 1)
    def _():
        ii = mi.astype(jnp.int32)
        vals_ref[...] = mv
        idx_ref[...] = ii
        e = jnp.exp(mv - mv[:, :1])
        w = e / jnp.sum(e, axis=-1, keepdims=True)
        par = (ii & 1).astype(jnp.float32)
        wlo_ref[...] = w * (1.0 - par)
        whi_ref[...] = w * par
        i2_ref[...] = lax.shift_right_logical(ii, 1)


def _tc_topk(queries, keys, kb):
    qq, d = queries.shape
    kk = keys.shape[0]
    assert kk % kb == 0
    nsteps = kk // kb
    out = pl.pallas_call(
        _tc_body,
        grid=(nsteps,),
        in_specs=[
            pl.BlockSpec((qq, d), lambda i: (0, 0)),
            pl.BlockSpec((kb, d), lambda i: (i, 0)),
        ],
        out_specs=[pl.BlockSpec((qq, TOPK), lambda i: (0, 0))] * 5,
        out_shape=[
            jax.ShapeDtypeStruct((qq, TOPK), jnp.float32),
            jax.ShapeDtypeStruct((qq, TOPK), jnp.int32),
            jax.ShapeDtypeStruct((qq, TOPK), jnp.float32),
            jax.ShapeDtypeStruct((qq, TOPK), jnp.float32),
            jax.ShapeDtypeStruct((qq, TOPK), jnp.int32),
        ],
        scratch_shapes=[
            pltpu.VMEM((qq, d), jnp.float32),
            pltpu.VMEM((qq, RUN), jnp.float32),
            pltpu.VMEM((qq, RUN), jnp.float32),
        ],
        compiler_params=pltpu.CompilerParams(
            vmem_limit_bytes=100 * 1024 * 1024),
    )(queries, keys)
    return out


_NW = 32
_QPW = 1024 // _NW
_GCH = 2
_NPC = _QPW // _GCH * TOPK


def _sc_body(keys2_hbm, wlo_hbm, whi_hbm, idx2_hbm, out_hbm,
             idx_v, wlo_v, whi_v, rows_v, acc_v, sem):
    wid = lax.axis_index("s") * 2 + lax.axis_index("c")
    qbase = wid * _QPW
    fbase = qbase * TOPK

    pltpu.sync_copy(idx2_hbm.at[pl.ds(wid * _GCH, _GCH)], idx_v)
    pltpu.sync_copy(wlo_hbm.at[pl.ds(fbase, _QPW * TOPK)], wlo_v)
    pltpu.sync_copy(whi_hbm.at[pl.ds(fbase, _QPW * TOPK)], whi_v)
    for cc in range(_GCH):
        pltpu.async_copy(keys2_hbm.at[idx_v.at[cc]], rows_v.at[cc], sem).wait()

    qph = _QPW // _GCH

    nw = _QPW * TOPK // 16
    wlregs = [wlo_v[pl.ds(r * 16, 16)] for r in range(nw)]
    whregs = [whi_v[pl.ds(r * 16, 16)] for r in range(nw)]

    for cc in range(_GCH):
        for qi in range(qph):
            q = cc * qph + qi
            for c in range(4):
                acc = jnp.zeros((16,), jnp.float32)
                for j in range(TOPK):
                    slot = q * TOPK + j
                    wl = wlregs[slot // 16][slot % 16]
                    wh = whregs[slot // 16][slot % 16]
                    row = rows_v.at[cc, qi * TOPK + j]
                    acc = acc + wl * row[pl.ds(c * 16, 16)]
                    acc = acc + wh * row[pl.ds(64 + c * 16, 16)]
                acc_v[q, pl.ds(c * 16, 16)] = acc

    pltpu.sync_copy(acc_v, out_hbm.at[pl.ds(qbase, _QPW)])


def _sc_aggregate(keys, wlo, whi, idx2):
    qq = wlo.shape[0]
    d = keys.shape[1]
    keys2 = keys.reshape(keys.shape[0] // 2, 2 * d)
    idx2r = idx2.reshape(_NW * _GCH, _NPC)
    mesh = plsc.VectorSubcoreMesh(core_axis_name="c", subcore_axis_name="s")
    run = pl.kernel(
        _sc_body,
        out_type=jax.ShapeDtypeStruct((qq, d), jnp.float32),
        mesh=mesh,
        scratch_types=[
            pltpu.VMEM((_GCH, _NPC), jnp.int32),
            pltpu.VMEM((_QPW * TOPK,), jnp.float32),
            pltpu.VMEM((_QPW * TOPK,), jnp.float32),
            pltpu.VMEM((_GCH, _NPC, 2 * d), jnp.float32),
            pltpu.VMEM((_QPW, d), jnp.float32),
            pltpu.SemaphoreType.DMA,
        ],
    )
    return run(keys2, wlo.reshape(-1), whi.reshape(-1), idx2r)


@jax.jit
def kernel(queries, keys):
    vals, idx, wlo, whi, idx2 = _tc_topk(queries, keys, kb=2000)
    aggregated = _sc_aggregate(keys, wlo, whi, idx2)
    return aggregated, vals, idx


# speedup vs baseline: 1.1981x; 1.1981x over previous
import functools

import jax
import jax.numpy as jnp
from jax import lax
from jax.experimental import pallas as pl
from jax.experimental.pallas import tpu as pltpu
from jax.experimental.pallas import tpu_sc as plsc

TOPK = 5
RUN = 8
NEG = float("-inf")
IMAX = 2**31 - 1


BIG = 1.0e8


def _tc_body(q_ref, k_ref, vals_ref, idx_ref, wlo_ref, whi_ref, i2_ref,
             qn_sc, rv_sc, ri_sc):
    step = pl.program_id(0)
    nsteps = pl.num_programs(0)
    kb = k_ref.shape[0]

    @pl.when(step == 0)
    def _():
        rv_sc[...] = jnp.full(rv_sc.shape, NEG, jnp.float32)
        ri_sc[...] = jnp.full(ri_sc.shape, BIG, jnp.float32)
        q = q_ref[...]
        qn_sc[...] = q / (jnp.sqrt(jnp.sum(q * q, axis=-1, keepdims=True))
                          + 1e-8)

    k = k_ref[...]
    kn = k / (jnp.sqrt(jnp.sum(k * k, axis=-1, keepdims=True)) + 1e-8)
    s = lax.dot_general(qn_sc[...], kn, (((1,), (1,)), ((), ())),
                        preferred_element_type=jnp.float32)
    fl = lax.broadcasted_iota(jnp.int32, s.shape, 1).astype(jnp.float32)
    base = (step * kb).astype(jnp.float32)
    rv0 = rv_sc[...]
    ri0 = ri_sc[...]

    x, rv = s, rv0
    vs, is_ = [], []
    for _ in range(TOPK):
        m = jnp.maximum(jnp.max(x, axis=-1, keepdims=True),
                        jnp.max(rv, axis=-1, keepdims=True))
        ib = jnp.min(jnp.where(x == m, fl, BIG), axis=-1, keepdims=True) + base
        ir = jnp.min(jnp.where(rv == m, ri0, BIG), axis=-1, keepdims=True)
        i = jnp.minimum(ib, ir)
        x = jnp.where(fl == i - base, NEG, x)
        rv = jnp.where(ri0 == i, NEG, rv)
        vs.append(m)
        is_.append(i)
    mv = jnp.concatenate(vs, axis=-1)
    mi = jnp.concatenate(is_, axis=-1)

    rv_sc[:, :TOPK] = mv
    ri_sc[:, :TOPK] = mi

    @pl.when(step == nsteps - 1)
    def _():
        ii = mi.astype(jnp.int32)
        vals_ref[...] = mv
        idx_ref[...] = ii
        e = jnp.exp(mv - mv[:, :1])
        w = e / jnp.sum(e, axis=-1, keepdims=True)
        par = (ii & 1).astype(jnp.float32)
        wlo_ref[...] = w * (1.0 - par)
        whi_ref[...] = w * par
        i2_ref[...] = lax.shift_right_logical(ii, 1)


def _tc_topk(queries, keys, kb):
    qq, d = queries.shape
    kk = keys.shape[0]
    assert kk % kb == 0
    nsteps = kk // kb
    out = pl.pallas_call(
        _tc_body,
        grid=(nsteps,),
        in_specs=[
            pl.BlockSpec((qq, d), lambda i: (0, 0)),
            pl.BlockSpec((kb, d), lambda i: (i, 0)),
        ],
        out_specs=[pl.BlockSpec((qq, TOPK), lambda i: (0, 0))] * 5,
        out_shape=[
            jax.ShapeDtypeStruct((qq, TOPK), jnp.float32),
            jax.ShapeDtypeStruct((qq, TOPK), jnp.int32),
            jax.ShapeDtypeStruct((qq, TOPK), jnp.float32),
            jax.ShapeDtypeStruct((qq, TOPK), jnp.float32),
            jax.ShapeDtypeStruct((qq, TOPK), jnp.int32),
        ],
        scratch_shapes=[
            pltpu.VMEM((qq, d), jnp.float32),
            pltpu.VMEM((qq, RUN), jnp.float32),
            pltpu.VMEM((qq, RUN), jnp.float32),
        ],
        compiler_params=pltpu.CompilerParams(
            vmem_limit_bytes=100 * 1024 * 1024),
    )(queries, keys)
    return out


_NW = 32
_QPW = 1024 // _NW
_GCH = 2
_NPC = _QPW // _GCH * TOPK


def _sc_body(keys2_hbm, wlo_hbm, whi_hbm, idx2_hbm, out_hbm,
             idx_v, wlo_v, whi_v, rows_v, acc_v, sem):
    wid = lax.axis_index("s") * 2 + lax.axis_index("c")
    qbase = wid * _QPW
    fbase = qbase * TOPK

    pltpu.sync_copy(idx2_hbm.at[pl.ds(wid * _GCH, _GCH)], idx_v)
    pltpu.sync_copy(wlo_hbm.at[pl.ds(fbase, _QPW * TOPK)], wlo_v)
    pltpu.sync_copy(whi_hbm.at[pl.ds(fbase, _QPW * TOPK)], whi_v)
    for cc in range(_GCH):
        pltpu.async_copy(keys2_hbm.at[idx_v.at[cc]], rows_v.at[cc], sem).wait()

    qph = _QPW // _GCH

    nw = _QPW * TOPK // 16
    wlregs = [wlo_v[pl.ds(r * 16, 16)] for r in range(nw)]
    whregs = [whi_v[pl.ds(r * 16, 16)] for r in range(nw)]

    for cc in range(_GCH):
        for qi in range(qph):
            q = cc * qph + qi
            for c in range(4):
                acc = jnp.zeros((16,), jnp.float32)
                for j in range(TOPK):
                    slot = q * TOPK + j
                    wl = wlregs[slot // 16][slot % 16]
                    wh = whregs[slot // 16][slot % 16]
                    row = rows_v.at[cc, qi * TOPK + j]
                    acc = acc + wl * row[pl.ds(c * 16, 16)]
                    acc = acc + wh * row[pl.ds(64 + c * 16, 16)]
                acc_v[q, pl.ds(c * 16, 16)] = acc

    pltpu.sync_copy(acc_v, out_hbm.at[pl.ds(qbase, _QPW)])


def _sc_aggregate(keys, wlo, whi, idx2):
    qq = wlo.shape[0]
    d = keys.shape[1]
    keys2 = keys.reshape(keys.shape[0] // 2, 2 * d)
    idx2r = idx2.reshape(_NW * _GCH, _NPC)
    mesh = plsc.VectorSubcoreMesh(core_axis_name="c", subcore_axis_name="s")
    run = pl.kernel(
        _sc_body,
        out_type=jax.ShapeDtypeStruct((qq, d), jnp.float32),
        mesh=mesh,
        scratch_types=[
            pltpu.VMEM((_GCH, _NPC), jnp.int32),
            pltpu.VMEM((_QPW * TOPK,), jnp.float32),
            pltpu.VMEM((_QPW * TOPK,), jnp.float32),
            pltpu.VMEM((_GCH, _NPC, 2 * d), jnp.float32),
            pltpu.VMEM((_QPW, d), jnp.float32),
            pltpu.SemaphoreType.DMA,
        ],
    )
    return run(keys2, wlo.reshape(-1), whi.reshape(-1), idx2r)


@jax.jit
def kernel(queries, keys):
    vals, idx, wlo, whi, idx2 = _tc_topk(queries, keys, kb=5000)
    aggregated = _sc_aggregate(keys, wlo, whi, idx2)
    return aggregated, vals, idx
